# baseline (speedup 1.0000x reference)
import jax
import jax.numpy as jnp
from jax import lax
from jax.experimental import pallas as pl
from jax.experimental.pallas import tpu as pltpu

_NUM_HEADS = 16
_MAX_DIST = 512
_SEQ = 2048
_R = 128
_NBLK = _SEQ // _R
_EXT = 2 * _SEQ


def _bias_kernel(tab_ref, out_ref, m_ref, sem_ref):
    h = pl.program_id(0)
    p = pl.program_id(1)
    slot = lax.rem(h, 2)

    def _dma(slot_idx, blk, head):
        off = _SEQ - _R - blk * _R
        return pltpu.make_async_copy(
            m_ref.at[slot_idx, :, pl.ds(off, _SEQ)],
            out_ref.at[head, pl.ds(blk * _R, _R), :],
            sem_ref.at[slot_idx],
        )

    @pl.when(p == 0)
    def _build_master():
        @pl.when(h >= 2)
        def _drain():
            @pl.loop(0, _NBLK)
            def _(b):
                _dma(slot, b, h - 2).wait()

        tcol = tab_ref[0, 0:1, 0:2 * _MAX_DIST + 1]
        t_lo = tab_ref[0, 0, 0]
        t_hi = tab_ref[0, 0, 2 * _MAX_DIST]
        lo_w = _SEQ - 1 - _MAX_DIST
        hi_w = _EXT - lo_w - (2 * _MAX_DIST + 1)
        ext = jnp.concatenate(
            [
                jnp.full((1, lo_w), t_lo, jnp.float32),
                tcol,
                jnp.full((1, hi_w), t_hi, jnp.float32),
            ],
            axis=1,
        )

        x = jnp.broadcast_to(ext, (_R, _EXT))
        rows = lax.broadcasted_iota(jnp.int32, (_R, 1), 0)
        rr = (_R - 1) - rows
        for k in range(7):
            m = 1 << k
            rolled = jnp.concatenate([x[:, m:], x[:, :m]], axis=1)
            x = jnp.where((rr >> k) & 1 == 1, rolled, x)
        m_ref[slot] = x

    _dma(slot, p, h).start()

    @pl.when((h == _NUM_HEADS - 1) & (p == _NBLK - 1))
    def _final_drain():
        @pl.loop(0, _NBLK)
        def _(b):
            _dma(1 - slot, b, h - 1).wait()

        @pl.loop(0, _NBLK)
        def _(b):
            _dma(slot, b, h).wait()


def _bias_pallas(table_t):
    return pl.pallas_call(
        _bias_kernel,
        grid=(_NUM_HEADS, _NBLK),
        in_specs=[
            pl.BlockSpec((1, 1, table_t.shape[2]), lambda h, p: (h, 0, 0)),
        ],
        out_specs=pl.BlockSpec(memory_space=pltpu.MemorySpace.HBM),
        out_shape=jax.ShapeDtypeStruct((_NUM_HEADS, _SEQ, _SEQ), jnp.float32),
        scratch_shapes=[
            pltpu.VMEM((2, _R, _EXT), jnp.float32),
            pltpu.SemaphoreType.DMA((2,)),
        ],
        compiler_params=pltpu.CompilerParams(
            dimension_semantics=("arbitrary", "arbitrary"),
        ),
    )(table_t)


def kernel(seq_len, table):
    table_t = jnp.pad(table.T, ((0, 0), (0, 127)))[:, None, :]
    return _bias_pallas(table_t)


# speedup vs baseline: 1.0061x; 1.0061x over previous
import jax
import jax.numpy as jnp
from jax import lax
from jax.experimental import pallas as pl
from jax.experimental.pallas import tpu as pltpu

_NUM_HEADS = 16
_MAX_DIST = 512
_SEQ = 2048
_R = 128
_NBLK = _SEQ // _R
_EXT = 2 * _SEQ


def _bias_kernel(tab_ref, out_ref, m_ref, sem_ref):
    h = pl.program_id(0)
    p = pl.program_id(1)
    slot = lax.rem(h, 2)

    def _dma(slot_idx, blk, head):
        off = _SEQ - _R - blk * _R
        return pltpu.make_async_copy(
            m_ref.at[slot_idx, :, pl.ds(off, _SEQ)],
            out_ref.at[head, pl.ds(blk * _R, _R), :],
            sem_ref.at[slot_idx, lax.rem(blk, 2)],
        )

    @pl.when(p == 0)
    def _build_master():
        @pl.when(h >= 2)
        def _drain():
            @pl.loop(0, _NBLK)
            def _(b):
                _dma(slot, b, h - 2).wait()

        tcol = tab_ref[0, 0:1, 0:2 * _MAX_DIST + 1]
        t_lo = tab_ref[0, 0, 0]
        t_hi = tab_ref[0, 0, 2 * _MAX_DIST]
        lo_w = _SEQ - 1 - _MAX_DIST
        hi_w = _EXT - lo_w - (2 * _MAX_DIST + 1)
        ext = jnp.concatenate(
            [
                jnp.full((1, lo_w), t_lo, jnp.float32),
                tcol,
                jnp.full((1, hi_w), t_hi, jnp.float32),
            ],
            axis=1,
        )

        x = jnp.broadcast_to(ext, (_R, _EXT))
        rows = lax.broadcasted_iota(jnp.int32, (_R, 1), 0)
        rr = (_R - 1) - rows
        for k in range(7):
            m = 1 << k
            rolled = jnp.concatenate([x[:, m:], x[:, :m]], axis=1)
            x = jnp.where((rr >> k) & 1 == 1, rolled, x)
        m_ref[slot] = x

    _dma(slot, p, h).start()

    @pl.when((h == _NUM_HEADS - 1) & (p == _NBLK - 1))
    def _final_drain():
        @pl.loop(0, _NBLK)
        def _(b):
            _dma(1 - slot, b, h - 1).wait()

        @pl.loop(0, _NBLK)
        def _(b):
            _dma(slot, b, h).wait()


def _bias_pallas(table_t):
    return pl.pallas_call(
        _bias_kernel,
        grid=(_NUM_HEADS, _NBLK),
        in_specs=[
            pl.BlockSpec((1, 1, table_t.shape[2]), lambda h, p: (h, 0, 0)),
        ],
        out_specs=pl.BlockSpec(memory_space=pltpu.MemorySpace.HBM),
        out_shape=jax.ShapeDtypeStruct((_NUM_HEADS, _SEQ, _SEQ), jnp.float32),
        scratch_shapes=[
            pltpu.VMEM((2, _R, _EXT), jnp.float32),
            pltpu.SemaphoreType.DMA((2, 2)),
        ],
        compiler_params=pltpu.CompilerParams(
            dimension_semantics=("arbitrary", "arbitrary"),
        ),
    )(table_t)


def kernel(seq_len, table):
    table_t = jnp.pad(table.T, ((0, 0), (0, 127)))[:, None, :]
    return _bias_pallas(table_t)
